# baseline (speedup 1.0000x reference)
import functools

import jax
import jax.numpy as jnp
from jax import lax
from jax.experimental import pallas as pl
from jax.experimental.pallas import tpu as pltpu
from jax.experimental.pallas import tpu_sc as plsc

_NC = 2
_NS = 16
_NW = _NC * _NS
_L = 16
_CPB = 8


@functools.lru_cache(maxsize=None)
def _build(V, NB, R, D):
    bpw = NB // _NW
    n_chunks = bpw // _CPB
    assert n_chunks % 2 == 0
    nfull = R // _L
    rem = R - nfull * _L
    mesh = plsc.VectorSubcoreMesh(core_axis_name="c", subcore_axis_name="s")

    @functools.partial(
        pl.kernel,
        mesh=mesh,
        out_type=jax.ShapeDtypeStruct((NB, R, D), jnp.float32),
        scratch_types=[
            pltpu.VMEM((bpw, R), jnp.int32),
            pltpu.VMEM((2, _CPB, R, D), jnp.float32),
            pltpu.SemaphoreType.DMA,
            pltpu.SemaphoreType.DMA,
            pltpu.SemaphoreType.DMA,
            pltpu.SemaphoreType.DMA,
        ],
    )
    def k(table_hbm, idx_hbm, out_hbm, idx_v, rows_v, g0, g1, w0, w1):
        gsems = (g0, g1)
        wsems = (w0, w1)
        wid = lax.axis_index("s") * _NC + lax.axis_index("c")
        base = wid * bpw
        pltpu.sync_copy(idx_hbm.at[pl.ds(base, bpw)], idx_v)

        def fire_seq(row, vbuf, sem):
            for v in range(nfull):
                vec = idx_v[row, pl.ds(v * _L, _L)]
                for u in range(_L):
                    pltpu.async_copy(table_hbm.at[vec[u]],
                                     vbuf.at[v * _L + u], sem)
            if rem:
                vec = idx_v[row, pl.ds(R - _L, _L)]
                for u in range(_L - rem, _L):
                    pltpu.async_copy(table_hbm.at[vec[u]],
                                     vbuf.at[R - _L + u], sem)

        def fire_rows(c, b):
            for br in range(_CPB):
                fire_seq(c * _CPB + br, rows_v.at[b, br], gsems[b])

        def drain_rows(b):
            for br in range(_CPB):
                pltpu.make_async_copy(
                    out_hbm.at[0], rows_v.at[b, br], gsems[b]).wait()

        def write_start(c, b):
            pltpu.async_copy(
                rows_v.at[b], out_hbm.at[pl.ds(base + c * _CPB, _CPB)],
                wsems[b])

        def write_wait(c, b):
            pltpu.make_async_copy(
                rows_v.at[b], out_hbm.at[pl.ds(base + c * _CPB, _CPB)],
                wsems[b]).wait()

        fire_rows(0, 0)

        def group(g, carry):
            for b in range(2):
                c = g * 2 + b
                nb = 1 - b

                @pl.when(c + 1 < n_chunks)
                def _():
                    @pl.when(c - 1 >= 0)
                    def _():
                        write_wait(c - 1, nb)

                    fire_rows(c + 1, nb)

                drain_rows(b)
                write_start(c, b)
            return carry

        lax.fori_loop(0, n_chunks // 2, group, 0)

        write_wait(n_chunks - 2, (n_chunks - 2) % 2)
        write_wait(n_chunks - 1, (n_chunks - 1) % 2)

    return k


def kernel(table, inputs):
    V, D = table.shape
    NB, R = inputs.shape
    idx = inputs.astype(jnp.int32)
    pad = (-NB) % (_NW * _CPB * 2)
    if pad:
        idx = jnp.pad(idx, ((0, pad), (0, 0)))
    out = _build(V, NB + pad, R, D)(table, idx)
    if pad:
        out = out[:NB]
    return out


# speedup vs baseline: 1.0040x; 1.0040x over previous
import functools

import jax
import jax.numpy as jnp
from jax import lax
from jax.experimental import pallas as pl
from jax.experimental.pallas import tpu as pltpu
from jax.experimental.pallas import tpu_sc as plsc

_NC = 2
_NS = 16
_NW = _NC * _NS
_L = 16
_CPB = 4


@functools.lru_cache(maxsize=None)
def _build(V, NB, R, D):
    bpw = NB // _NW
    n_chunks = bpw // _CPB
    assert n_chunks % 2 == 0
    nfull = R // _L
    rem = R - nfull * _L
    mesh = plsc.VectorSubcoreMesh(core_axis_name="c", subcore_axis_name="s")

    @functools.partial(
        pl.kernel,
        mesh=mesh,
        out_type=jax.ShapeDtypeStruct((NB, R, D), jnp.float32),
        scratch_types=[
            pltpu.VMEM((bpw, R), jnp.int32),
            pltpu.VMEM((2, _CPB, R, D), jnp.float32),
            pltpu.SemaphoreType.DMA,
            pltpu.SemaphoreType.DMA,
            pltpu.SemaphoreType.DMA,
            pltpu.SemaphoreType.DMA,
        ],
    )
    def k(table_hbm, idx_hbm, out_hbm, idx_v, rows_v, g0, g1, w0, w1):
        gsems = (g0, g1)
        wsems = (w0, w1)
        wid = lax.axis_index("s") * _NC + lax.axis_index("c")
        base = wid * bpw
        pltpu.sync_copy(idx_hbm.at[pl.ds(base, bpw)], idx_v)

        def fire_seq(row, vbuf, sem):
            for v in range(nfull):
                vec = idx_v[row, pl.ds(v * _L, _L)]
                for u in range(_L):
                    pltpu.async_copy(table_hbm.at[vec[u]],
                                     vbuf.at[v * _L + u], sem)
            if rem:
                vec = idx_v[row, pl.ds(R - _L, _L)]
                for u in range(_L - rem, _L):
                    pltpu.async_copy(table_hbm.at[vec[u]],
                                     vbuf.at[R - _L + u], sem)

        def fire_rows(c, b):
            for br in range(_CPB):
                fire_seq(c * _CPB + br, rows_v.at[b, br], gsems[b])

        def drain_rows(b):
            for br in range(_CPB):
                pltpu.make_async_copy(
                    out_hbm.at[0], rows_v.at[b, br], gsems[b]).wait()

        def write_start(c, b):
            pltpu.async_copy(
                rows_v.at[b], out_hbm.at[pl.ds(base + c * _CPB, _CPB)],
                wsems[b])

        def write_wait(c, b):
            pltpu.make_async_copy(
                rows_v.at[b], out_hbm.at[pl.ds(base + c * _CPB, _CPB)],
                wsems[b]).wait()

        fire_rows(0, 0)

        def group(g, carry):
            for b in range(2):
                c = g * 2 + b
                nb = 1 - b

                @pl.when(c + 1 < n_chunks)
                def _():
                    @pl.when(c - 1 >= 0)
                    def _():
                        write_wait(c - 1, nb)

                    fire_rows(c + 1, nb)

                drain_rows(b)
                write_start(c, b)
            return carry

        lax.fori_loop(0, n_chunks // 2, group, 0)

        write_wait(n_chunks - 2, (n_chunks - 2) % 2)
        write_wait(n_chunks - 1, (n_chunks - 1) % 2)

    return k


def kernel(table, inputs):
    V, D = table.shape
    NB, R = inputs.shape
    idx = inputs.astype(jnp.int32)
    pad = (-NB) % (_NW * _CPB * 2)
    if pad:
        idx = jnp.pad(idx, ((0, pad), (0, 0)))
    out = _build(V, NB + pad, R, D)(table, idx)
    if pad:
        out = out[:NB]
    return out
